# baseline (speedup 1.0000x reference)
import functools

import jax
import jax.numpy as jnp
from jax import lax
from jax.experimental import pallas as pl
from jax.experimental.pallas import tpu as pltpu
from jax.experimental.pallas import tpu_sc as plsc


def _sc_permute(xf, perm, R, F):
    info = plsc.get_sparse_core_info()
    NC, NS, L = info.num_cores, info.num_subcores, info.num_lanes
    NW = NC * NS
    rows_per_w = R // NW
    BLK = 8
    n_blocks = rows_per_w // BLK
    blk_elems = BLK * F

    mesh = plsc.VectorSubcoreMesh(core_axis_name="c", subcore_axis_name="s")

    @functools.partial(
        pl.kernel,
        mesh=mesh,
        out_type=jax.ShapeDtypeStruct((R * F,), jnp.float32),
        scratch_types=[
            pltpu.VMEM((F,), jnp.int32),
            pltpu.VMEM((blk_elems,), jnp.float32),
            pltpu.VMEM((blk_elems,), jnp.float32),
        ],
        compiler_params=pltpu.CompilerParams(needs_layout_passes=False),
    )
    def run(x_hbm, perm_hbm, out_hbm, idx_v, in_v, out_v):
        wid = lax.axis_index("s") * NC + lax.axis_index("c")
        pltpu.sync_copy(perm_hbm, idx_v)
        base = wid * (rows_per_w * F)

        def blk_body(b, carry):
            off = base + b * blk_elems
            pltpu.sync_copy(x_hbm.at[pl.ds(off, blk_elems)], in_v)

            def g_body(g, carry2):
                idx = idx_v[pl.ds(g * L, L)]
                for r in range(BLK):
                    vals = plsc.load_gather(in_v, [idx + r * F])
                    out_v[pl.ds(r * F + g * L, L)] = vals
                return carry2

            lax.fori_loop(0, F // L, g_body, 0, unroll=2)
            pltpu.sync_copy(out_v, out_hbm.at[pl.ds(off, blk_elems)])
            return carry

        lax.fori_loop(0, n_blocks, blk_body, 0)

    return run(xf, perm)


def kernel(x, perm):
    B, S, F = x.shape
    R = B * S
    out = _sc_permute(x.reshape(R * F), perm, R, F)
    return out.reshape(B, S, F)


# speedup vs baseline: 1.2340x; 1.2340x over previous
import functools

import jax
import jax.numpy as jnp
from jax import lax
from jax.experimental import pallas as pl
from jax.experimental.pallas import tpu as pltpu
from jax.experimental.pallas import tpu_sc as plsc


def _sc_permute(xf, perm, R, F):
    info = plsc.get_sparse_core_info()
    NC, NS, L = info.num_cores, info.num_subcores, info.num_lanes
    NW = NC * NS
    rows_per_w = R // NW
    BLK = 8
    n_blocks = rows_per_w // BLK
    blk_elems = BLK * F

    mesh = plsc.VectorSubcoreMesh(core_axis_name="c", subcore_axis_name="s")

    @functools.partial(
        pl.kernel,
        mesh=mesh,
        out_type=jax.ShapeDtypeStruct((R * F,), jnp.float32),
        scratch_types=[
            pltpu.VMEM((F,), jnp.int32),
            pltpu.VMEM((blk_elems,), jnp.float32),
            pltpu.VMEM((blk_elems,), jnp.float32),
            pltpu.VMEM((blk_elems,), jnp.float32),
            pltpu.VMEM((blk_elems,), jnp.float32),
            pltpu.SemaphoreType.DMA,
            pltpu.SemaphoreType.DMA,
            pltpu.SemaphoreType.DMA,
            pltpu.SemaphoreType.DMA,
        ],
        compiler_params=pltpu.CompilerParams(needs_layout_passes=False),
    )
    def run(x_hbm, perm_hbm, out_hbm, idx_v, in0, in1, out0, out1,
            si0, si1, so0, so1):
        wid = lax.axis_index("s") * NC + lax.axis_index("c")
        base = wid * (rows_per_w * F)
        ins, outs = (in0, in1), (out0, out1)
        isems, osems = (si0, si1), (so0, so1)

        def in_copy(b, ph):
            return pltpu.make_async_copy(
                x_hbm.at[pl.ds(base + b * blk_elems, blk_elems)],
                ins[ph], isems[ph])

        def out_copy(b, ph):
            return pltpu.make_async_copy(
                outs[ph], out_hbm.at[pl.ds(base + b * blk_elems, blk_elems)],
                osems[ph])

        pltpu.sync_copy(perm_hbm, idx_v)
        in_copy(0, 0).start()
        in_copy(1, 1).start()

        def pair_body(bb, carry):
            for ph in (0, 1):
                b = bb * 2 + ph

                @pl.when(bb > 0)
                def _():
                    out_copy(b - 2, ph).wait()

                in_copy(b, ph).wait()
                in_v, out_v = ins[ph], outs[ph]

                def g_body(g, c2):
                    idx = idx_v[pl.ds(g * L, L)]
                    for r in range(BLK):
                        vals = plsc.load_gather(in_v, [idx + r * F])
                        out_v[pl.ds(r * F + g * L, L)] = vals
                    return c2

                lax.fori_loop(0, F // L, g_body, 0, unroll=4)
                out_copy(b, ph).start()

                @pl.when(b + 2 < n_blocks)
                def _():
                    in_copy(b + 2, ph).start()
            return carry

        lax.fori_loop(0, n_blocks // 2, pair_body, 0)
        out_copy(n_blocks - 2, 0).wait()
        out_copy(n_blocks - 1, 1).wait()

    return run(xf, perm)


def kernel(x, perm):
    B, S, F = x.shape
    R = B * S
    out = _sc_permute(x.reshape(R * F), perm, R, F)
    return out.reshape(B, S, F)


# speedup vs baseline: 5.9311x; 4.8063x over previous
import functools

import jax
import jax.numpy as jnp
from jax import lax
from jax.experimental import pallas as pl
from jax.experimental.pallas import tpu as pltpu
from jax.experimental.pallas import tpu_sc as plsc


def kernel(x, perm):
    B, S, F = x.shape
    info = plsc.get_sparse_core_info()
    NC, NS, L = info.num_cores, info.num_subcores, info.num_lanes
    NW = NC * NS
    R = B * S
    rows_per_w = R // NW
    w_per_b = S // rows_per_w
    BLK = 8
    n_blocks = rows_per_w // BLK

    mesh = plsc.VectorSubcoreMesh(core_axis_name="c", subcore_axis_name="s")

    @functools.partial(
        pl.kernel,
        mesh=mesh,
        out_type=jax.ShapeDtypeStruct((B, S, F), jnp.float32),
        scratch_types=[
            pltpu.VMEM((F,), jnp.int32),
            pltpu.VMEM((BLK, F), jnp.float32),
            pltpu.VMEM((BLK, F), jnp.float32),
            pltpu.VMEM((BLK, F), jnp.float32),
            pltpu.VMEM((BLK, F), jnp.float32),
            pltpu.SemaphoreType.DMA,
            pltpu.SemaphoreType.DMA,
            pltpu.SemaphoreType.DMA,
            pltpu.SemaphoreType.DMA,
        ],
        compiler_params=pltpu.CompilerParams(needs_layout_passes=False),
    )
    def run(x_hbm, perm_hbm, out_hbm, idx_v, in0, in1, out0, out1,
            si0, si1, so0, so1):
        wid = lax.axis_index("s") * NC + lax.axis_index("c")
        bidx = wid // w_per_b
        row0 = (wid % w_per_b) * rows_per_w
        ins, outs = (in0, in1), (out0, out1)
        isems, osems = (si0, si1), (so0, so1)

        def in_copy(b, ph):
            return pltpu.make_async_copy(
                x_hbm.at[bidx, pl.ds(row0 + b * BLK, BLK)],
                ins[ph], isems[ph])

        def out_copy(b, ph):
            return pltpu.make_async_copy(
                outs[ph], out_hbm.at[bidx, pl.ds(row0 + b * BLK, BLK)],
                osems[ph])

        pltpu.sync_copy(perm_hbm, idx_v)
        in_copy(0, 0).start()
        in_copy(1, 1).start()
        rows = [jnp.full((L,), r, jnp.int32) for r in range(BLK)]

        def pair_body(bb, carry):
            for ph in (0, 1):
                b = bb * 2 + ph

                @pl.when(bb > 0)
                def _():
                    out_copy(b - 2, ph).wait()

                in_copy(b, ph).wait()
                in_v, out_v = ins[ph], outs[ph]

                @plsc.parallel_loop(0, F // L, unroll=4)
                def _(g):
                    idx = idx_v[pl.ds(g * L, L)]
                    for r in range(BLK):
                        vals = plsc.load_gather(in_v, [rows[r], idx])
                        out_v[r, pl.ds(g * L, L)] = vals

                out_copy(b, ph).start()

                @pl.when(b + 2 < n_blocks)
                def _():
                    in_copy(b + 2, ph).start()
            return carry

        lax.fori_loop(0, n_blocks // 2, pair_body, 0)
        out_copy(n_blocks - 2, 0).wait()
        out_copy(n_blocks - 1, 1).wait()

    return run(x, perm)
